# baseline (speedup 1.0000x reference)
import functools

import jax
import jax.numpy as jnp
from jax import lax
from jax.experimental import pallas as pl
from jax.experimental.pallas import tpu as pltpu
from jax.experimental.pallas import tpu_sc as plsc

_MAX_L = 2
_NWAVE = 8
_N_NODES = 10000
_N_EDGES = 640000
_LANES = 16
_CHUNK = 8000


_NPARTS = 4
_NQUADS = 8
_EDGES_PER_PART = _N_EDGES // _NPARTS
_CHUNKS_PER_PART = _EDGES_PER_PART // _CHUNK


def _sc_scatter_body(coeff_hbm, pack_hbm, cart_hbm, out_hbm,
                     c0, c1, c2, c3, a0, a1, a2, a3,
                     bp0, bp1, mb0, mb1, sem0, sem1):
    nc = plsc.get_sparse_core_info().num_cores
    wid = lax.axis_index("s") * nc + lax.axis_index("c")

    q = wid & 7
    p = wid >> 3
    is_dis = q < 2
    col_base = jnp.where(is_dis, 4 * q, 9 + 4 * ((q - 2) % 2))
    rsel = jnp.where(is_dis, 0, (q - 2) // 2)
    ch_base = jnp.where(is_dis, 4 * q, 8 + 8 * ((q - 2) // 2)
                        + 4 * ((q - 2) % 2))
    g0 = p * _CHUNKS_PER_PART

    cols = (c0, c1, c2, c3)
    accs = (a0, a1, a2, a3)
    bufs = ((bp0, mb0, sem0), (bp1, mb1, sem1))

    def _fire(g, bp, mb, sem, use_mult):
        pltpu.async_copy(pack_hbm.at[pl.ds(g * 2 * _CHUNK, 2 * _CHUNK)],
                         bp, sem)
        if use_mult:
            pltpu.async_copy(
                cart_hbm.at[pl.ds(rsel * _N_EDGES + g * _CHUNK, _CHUNK)],
                mb, sem)

    def _drain(g, bp, mb, sem, use_mult):
        pltpu.make_async_copy(
            pack_hbm.at[pl.ds(g * 2 * _CHUNK, 2 * _CHUNK)], bp, sem).wait()
        if use_mult:
            pltpu.make_async_copy(
                cart_hbm.at[pl.ds(rsel * _N_EDGES + g * _CHUNK, _CHUNK)],
                mb, sem).wait()

    def _run_part(use_mult):
        _fire(g0, *bufs[0], use_mult)
        _fire(g0 + 1, *bufs[1], use_mult)

        def _chunk_pair(gg, _):
            for b in range(2):
                g = g0 + gg * 2 + b
                bp, mb, sem = bufs[b]
                _drain(g, bp, mb, sem, use_mult)

                @plsc.parallel_loop(0, _CHUNK // _LANES, unroll=16)
                def _vec(i):
                    s = pl.ds(i * _LANES, _LANES)
                    v = bp[s]
                    ni = v & 0xFFFF
                    ci = v >> 16
                    cutv = plsc.bitcast(
                        bp[pl.ds(_CHUNK + i * _LANES, _LANES)], jnp.float32)
                    w = cutv * mb[s] if use_mult else cutv
                    for l in range(4):
                        g16 = plsc.load_gather(cols[l], [ni])
                        plsc.addupdate_scatter(accs[l], [ci], w * g16)

                @pl.when(gg * 2 + b + 2 < _CHUNKS_PER_PART)
                def _():
                    _fire(g + 2, bp, mb, sem, use_mult)
            return 0

        lax.fori_loop(0, _CHUNKS_PER_PART // 2, _chunk_pair, 0)

    for l in range(4):
        pltpu.sync_copy(coeff_hbm.at[pl.ds((col_base + l) * _N_NODES,
                                           _N_NODES)], cols[l])

    @plsc.parallel_loop(0, _N_NODES // _LANES, unroll=8)
    def _zero(i):
        z = jnp.zeros((_LANES,), jnp.float32)
        for l in range(4):
            accs[l][pl.ds(i * _LANES, _LANES)] = z

    @pl.when(is_dis)
    def _():
        _run_part(False)

    @pl.when(jnp.logical_not(is_dis))
    def _():
        _run_part(True)

    for l in range(4):
        pltpu.sync_copy(
            accs[l],
            out_hbm.at[pl.ds((p * 32 + ch_base + l) * _N_NODES, _N_NODES)])


def _sc_scatter(coeff_t, pack, cart_t):
    mesh = plsc.VectorSubcoreMesh(core_axis_name="c", subcore_axis_name="s")
    f = pl.kernel(
        _sc_scatter_body,
        out_type=jax.ShapeDtypeStruct((_NPARTS * 32 * _N_NODES,),
                                      jnp.float32),
        mesh=mesh,
        scratch_types=(
            [pltpu.VMEM((_N_NODES,), jnp.float32)] * 4
            + [pltpu.VMEM((_N_NODES,), jnp.float32)] * 4
            + [pltpu.VMEM((2 * _CHUNK,), jnp.int32)] * 2
            + [pltpu.VMEM((_CHUNK,), jnp.float32)] * 2
            + [pltpu.SemaphoreType.DMA] * 2
        ),
        compiler_params=pltpu.CompilerParams(needs_layout_passes=False),
    )
    return f(coeff_t.reshape(-1), pack, cart_t.reshape(-1)).reshape(
        _NPARTS, 32, _N_NODES)


def _tc_stage2_body(acc_ref, scale_ref, mp_dis_ref, mp_cart_ref,
                    w1t_ref, b1_ref, w2t_ref, b2_ref, out_ref):
    a4 = acc_ref[...]
    acc = a4[0] + a4[1] + a4[2] + a4[3]
    sd = scale_ref[0:1, :]
    sc = scale_ref[1:2, :]

    md = (acc[0:8, :] + mp_dis_ref[...]) * sd
    h = jnp.dot(w1t_ref[...], md,
                preferred_element_type=jnp.float32) + b1_ref[...]
    h = h * (1.0 / (1.0 + jnp.exp(-h)))
    radial = jnp.dot(w2t_ref[...], h,
                     preferred_element_type=jnp.float32) + b2_ref[...]

    mc = (acc[8:32, :] + mp_cart_ref[...]) * sc
    x = mc[0:8, :]
    y = mc[8:16, :]
    z = mc[16:24, :]
    r2 = x * x + y * y + z * z
    s4 = x * y
    s5 = y * z
    s6 = 3.0 * z * z - r2
    s7 = x * z
    s8 = x * x - y * y
    ang2 = s4 * s4 + s5 * s5 + s6 * s6 + s7 * s7 + s8 * s8

    out_ref[0:8, :] = radial
    out_ref[8:16, :] = radial * r2
    out_ref[16:24, :] = radial * ang2


def _tc_stage2(acc, scale, mp_dis_t, mp_cart_f, w1t, b1c, w2t, b2c):
    return pl.pallas_call(
        _tc_stage2_body,
        out_shape=jax.ShapeDtypeStruct((24, _N_NODES), jnp.float32),
    )(acc, scale, mp_dis_t, mp_cart_f, w1t, b1c, w2t, b2c)


@jax.jit
def kernel(cart, cut_distances, iter_coeff, index_center, index_neigh,
           MP_dis, MP_cart, W1, b1, W2, b2):
    n = iter_coeff.shape[0]
    e = cart.shape[0]
    g = e // _CHUNK
    coeff_t = iter_coeff.T
    pidx = jnp.bitwise_or(index_neigh,
                          jnp.left_shift(index_center, 16))
    cutb = lax.bitcast_convert_type(cut_distances, jnp.int32)
    pack = jnp.stack([pidx.reshape(g, _CHUNK),
                      cutb.reshape(g, _CHUNK)], axis=1).reshape(-1)

    acc = _sc_scatter(coeff_t, pack, cart.T)

    scale = jnp.stack([iter_coeff[:, _NWAVE], iter_coeff[:, -1]], axis=0)
    mp_dis_t = MP_dis.T
    mp_cart_f = MP_cart.reshape(24, n)
    dens = _tc_stage2(acc, scale, mp_dis_t, mp_cart_f,
                      W1.T, b1[:, None], W2.T, b2[:, None])
    return dens.reshape(_MAX_L + 1, _NWAVE, n).transpose(2, 0, 1)


# speedup vs baseline: 1.0443x; 1.0443x over previous
import functools

import jax
import jax.numpy as jnp
from jax import lax
from jax.experimental import pallas as pl
from jax.experimental.pallas import tpu as pltpu
from jax.experimental.pallas import tpu_sc as plsc

_MAX_L = 2
_NWAVE = 8
_N_NODES = 10000
_N_EDGES = 640000
_LANES = 16
_CHUNK = 8000


_NPARTS = 4
_NQUADS = 8
_EDGES_PER_PART = _N_EDGES // _NPARTS
_CHUNKS_PER_PART = _EDGES_PER_PART // _CHUNK


def _sc_scatter_body(coeff_hbm, pack_hbm, cart_hbm, out_hbm,
                     c0, c1, c2, c3, a0, a1, a2, a3,
                     bp0, bp1, mb0, mb1, sem0, sem1):
    nc = plsc.get_sparse_core_info().num_cores
    wid = lax.axis_index("s") * nc + lax.axis_index("c")

    q = wid & 7
    p = wid >> 3
    is_dis = q < 2
    col_base = jnp.where(is_dis, 4 * q, 9 + 4 * ((q - 2) % 2))
    rsel = jnp.where(is_dis, 0, (q - 2) // 2)
    ch_base = jnp.where(is_dis, 4 * q, 8 + 8 * ((q - 2) // 2)
                        + 4 * ((q - 2) % 2))
    g0 = p * _CHUNKS_PER_PART

    cols = (c0, c1, c2, c3)
    accs = (a0, a1, a2, a3)
    bufs = ((bp0, mb0, sem0), (bp1, mb1, sem1))

    def _fire(g, bp, mb, sem, use_mult):
        pltpu.async_copy(pack_hbm.at[pl.ds(g * 2 * _CHUNK, 2 * _CHUNK)],
                         bp, sem)
        if use_mult:
            pltpu.async_copy(
                cart_hbm.at[pl.ds(rsel * _N_EDGES + g * _CHUNK, _CHUNK)],
                mb, sem)

    def _drain(g, bp, mb, sem, use_mult):
        pltpu.make_async_copy(
            pack_hbm.at[pl.ds(g * 2 * _CHUNK, 2 * _CHUNK)], bp, sem).wait()
        if use_mult:
            pltpu.make_async_copy(
                cart_hbm.at[pl.ds(rsel * _N_EDGES + g * _CHUNK, _CHUNK)],
                mb, sem).wait()

    def _run_part(use_mult):
        _fire(g0, *bufs[0], use_mult)
        _fire(g0 + 1, *bufs[1], use_mult)

        def _chunk_pair(gg, _):
            for b in range(2):
                g = g0 + gg * 2 + b
                bp, mb, sem = bufs[b]
                _drain(g, bp, mb, sem, use_mult)

                @plsc.parallel_loop(0, _CHUNK // _LANES, unroll=4)
                def _vec(i):
                    s = pl.ds(i * _LANES, _LANES)
                    v = bp[s]
                    ni = v & 0xFFFF
                    ci = v >> 16
                    cutv = plsc.bitcast(
                        bp[pl.ds(_CHUNK + i * _LANES, _LANES)], jnp.float32)
                    w = cutv * mb[s] if use_mult else cutv
                    for l in range(4):
                        g16 = plsc.load_gather(cols[l], [ni])
                        plsc.addupdate_scatter(accs[l], [ci], w * g16)

                @pl.when(gg * 2 + b + 2 < _CHUNKS_PER_PART)
                def _():
                    _fire(g + 2, bp, mb, sem, use_mult)
            return 0

        lax.fori_loop(0, _CHUNKS_PER_PART // 2, _chunk_pair, 0)

    for l in range(4):
        pltpu.sync_copy(coeff_hbm.at[pl.ds((col_base + l) * _N_NODES,
                                           _N_NODES)], cols[l])

    @plsc.parallel_loop(0, _N_NODES // _LANES, unroll=8)
    def _zero(i):
        z = jnp.zeros((_LANES,), jnp.float32)
        for l in range(4):
            accs[l][pl.ds(i * _LANES, _LANES)] = z

    @pl.when(is_dis)
    def _():
        _run_part(False)

    @pl.when(jnp.logical_not(is_dis))
    def _():
        _run_part(True)

    for l in range(4):
        pltpu.sync_copy(
            accs[l],
            out_hbm.at[pl.ds((p * 32 + ch_base + l) * _N_NODES, _N_NODES)])


def _sc_scatter(coeff_t, pack, cart_t):
    mesh = plsc.VectorSubcoreMesh(core_axis_name="c", subcore_axis_name="s")
    f = pl.kernel(
        _sc_scatter_body,
        out_type=jax.ShapeDtypeStruct((_NPARTS * 32 * _N_NODES,),
                                      jnp.float32),
        mesh=mesh,
        scratch_types=(
            [pltpu.VMEM((_N_NODES,), jnp.float32)] * 4
            + [pltpu.VMEM((_N_NODES,), jnp.float32)] * 4
            + [pltpu.VMEM((2 * _CHUNK,), jnp.int32)] * 2
            + [pltpu.VMEM((_CHUNK,), jnp.float32)] * 2
            + [pltpu.SemaphoreType.DMA] * 2
        ),
        compiler_params=pltpu.CompilerParams(needs_layout_passes=False),
    )
    return f(coeff_t.reshape(-1), pack, cart_t.reshape(-1)).reshape(
        _NPARTS, 32, _N_NODES)


def _tc_stage2_body(acc_ref, scale_ref, mp_dis_ref, mp_cart_ref,
                    w1t_ref, b1_ref, w2t_ref, b2_ref, out_ref):
    a4 = acc_ref[...]
    acc = a4[0] + a4[1] + a4[2] + a4[3]
    sd = scale_ref[0:1, :]
    sc = scale_ref[1:2, :]

    md = (acc[0:8, :] + mp_dis_ref[...]) * sd
    h = jnp.dot(w1t_ref[...], md,
                preferred_element_type=jnp.float32) + b1_ref[...]
    h = h * (1.0 / (1.0 + jnp.exp(-h)))
    radial = jnp.dot(w2t_ref[...], h,
                     preferred_element_type=jnp.float32) + b2_ref[...]

    mc = (acc[8:32, :] + mp_cart_ref[...]) * sc
    x = mc[0:8, :]
    y = mc[8:16, :]
    z = mc[16:24, :]
    r2 = x * x + y * y + z * z
    s4 = x * y
    s5 = y * z
    s6 = 3.0 * z * z - r2
    s7 = x * z
    s8 = x * x - y * y
    ang2 = s4 * s4 + s5 * s5 + s6 * s6 + s7 * s7 + s8 * s8

    out_ref[0:8, :] = radial
    out_ref[8:16, :] = radial * r2
    out_ref[16:24, :] = radial * ang2


def _tc_stage2(acc, scale, mp_dis_t, mp_cart_f, w1t, b1c, w2t, b2c):
    return pl.pallas_call(
        _tc_stage2_body,
        out_shape=jax.ShapeDtypeStruct((24, _N_NODES), jnp.float32),
    )(acc, scale, mp_dis_t, mp_cart_f, w1t, b1c, w2t, b2c)


@jax.jit
def kernel(cart, cut_distances, iter_coeff, index_center, index_neigh,
           MP_dis, MP_cart, W1, b1, W2, b2):
    n = iter_coeff.shape[0]
    e = cart.shape[0]
    g = e // _CHUNK
    coeff_t = iter_coeff.T
    pidx = jnp.bitwise_or(index_neigh,
                          jnp.left_shift(index_center, 16))
    cutb = lax.bitcast_convert_type(cut_distances, jnp.int32)
    pack = jnp.stack([pidx.reshape(g, _CHUNK),
                      cutb.reshape(g, _CHUNK)], axis=1).reshape(-1)

    acc = _sc_scatter(coeff_t, pack, cart.T)

    scale = jnp.stack([iter_coeff[:, _NWAVE], iter_coeff[:, -1]], axis=0)
    mp_dis_t = MP_dis.T
    mp_cart_f = MP_cart.reshape(24, n)
    dens = _tc_stage2(acc, scale, mp_dis_t, mp_cart_f,
                      W1.T, b1[:, None], W2.T, b2[:, None])
    return dens.reshape(_MAX_L + 1, _NWAVE, n).transpose(2, 0, 1)


# speedup vs baseline: 1.0513x; 1.0067x over previous
import functools

import jax
import jax.numpy as jnp
from jax import lax
from jax.experimental import pallas as pl
from jax.experimental.pallas import tpu as pltpu
from jax.experimental.pallas import tpu_sc as plsc

_MAX_L = 2
_NWAVE = 8
_N_NODES = 10000
_N_EDGES = 640000
_LANES = 16
_CHUNK = 8000


_NPARTS = 4
_NQUADS = 8
_EDGES_PER_PART = _N_EDGES // _NPARTS
_CHUNKS_PER_PART = _EDGES_PER_PART // _CHUNK


def _sc_scatter_body(coeff_hbm, pack_hbm, cart_hbm, out_hbm,
                     c0, c1, c2, c3, a0, a1, a2, a3,
                     bp0, bp1, mb0, mb1, sem0, sem1):
    nc = plsc.get_sparse_core_info().num_cores
    wid = lax.axis_index("s") * nc + lax.axis_index("c")

    q = wid & 7
    p = wid >> 3
    is_dis = q < 2
    col_base = jnp.where(is_dis, 4 * q, 9 + 4 * ((q - 2) % 2))
    rsel = jnp.where(is_dis, 0, (q - 2) // 2)
    ch_base = jnp.where(is_dis, 4 * q, 8 + 8 * ((q - 2) // 2)
                        + 4 * ((q - 2) % 2))
    g0 = p * _CHUNKS_PER_PART

    cols = (c0, c1, c2, c3)
    accs = (a0, a1, a2, a3)
    bufs = ((bp0, mb0, sem0), (bp1, mb1, sem1))

    def _fire(g, bp, mb, sem, use_mult):
        pltpu.async_copy(pack_hbm.at[pl.ds(g * 2 * _CHUNK, 2 * _CHUNK)],
                         bp, sem)
        if use_mult:
            pltpu.async_copy(
                cart_hbm.at[pl.ds(rsel * _N_EDGES + g * _CHUNK, _CHUNK)],
                mb, sem)

    def _drain(g, bp, mb, sem, use_mult):
        pltpu.make_async_copy(
            pack_hbm.at[pl.ds(g * 2 * _CHUNK, 2 * _CHUNK)], bp, sem).wait()
        if use_mult:
            pltpu.make_async_copy(
                cart_hbm.at[pl.ds(rsel * _N_EDGES + g * _CHUNK, _CHUNK)],
                mb, sem).wait()

    def _run_part(use_mult):
        _fire(g0, *bufs[0], use_mult)
        _fire(g0 + 1, *bufs[1], use_mult)

        def _chunk_pair(gg, _):
            for b in range(2):
                g = g0 + gg * 2 + b
                bp, mb, sem = bufs[b]
                _drain(g, bp, mb, sem, use_mult)

                @plsc.parallel_loop(0, _CHUNK // _LANES, unroll=2)
                def _vec(i):
                    s = pl.ds(i * _LANES, _LANES)
                    v = bp[s]
                    ni = v & 0xFFFF
                    ci = v >> 16
                    cutv = plsc.bitcast(
                        bp[pl.ds(_CHUNK + i * _LANES, _LANES)], jnp.float32)
                    w = cutv * mb[s] if use_mult else cutv
                    for l in range(4):
                        g16 = plsc.load_gather(cols[l], [ni])
                        plsc.addupdate_scatter(accs[l], [ci], w * g16)

                @pl.when(gg * 2 + b + 2 < _CHUNKS_PER_PART)
                def _():
                    _fire(g + 2, bp, mb, sem, use_mult)
            return 0

        lax.fori_loop(0, _CHUNKS_PER_PART // 2, _chunk_pair, 0)

    for l in range(4):
        pltpu.sync_copy(coeff_hbm.at[pl.ds((col_base + l) * _N_NODES,
                                           _N_NODES)], cols[l])

    @plsc.parallel_loop(0, _N_NODES // _LANES, unroll=8)
    def _zero(i):
        z = jnp.zeros((_LANES,), jnp.float32)
        for l in range(4):
            accs[l][pl.ds(i * _LANES, _LANES)] = z

    @pl.when(is_dis)
    def _():
        _run_part(False)

    @pl.when(jnp.logical_not(is_dis))
    def _():
        _run_part(True)

    for l in range(4):
        pltpu.sync_copy(
            accs[l],
            out_hbm.at[pl.ds((p * 32 + ch_base + l) * _N_NODES, _N_NODES)])


def _sc_scatter(coeff_t, pack, cart_t):
    mesh = plsc.VectorSubcoreMesh(core_axis_name="c", subcore_axis_name="s")
    f = pl.kernel(
        _sc_scatter_body,
        out_type=jax.ShapeDtypeStruct((_NPARTS * 32 * _N_NODES,),
                                      jnp.float32),
        mesh=mesh,
        scratch_types=(
            [pltpu.VMEM((_N_NODES,), jnp.float32)] * 4
            + [pltpu.VMEM((_N_NODES,), jnp.float32)] * 4
            + [pltpu.VMEM((2 * _CHUNK,), jnp.int32)] * 2
            + [pltpu.VMEM((_CHUNK,), jnp.float32)] * 2
            + [pltpu.SemaphoreType.DMA] * 2
        ),
        compiler_params=pltpu.CompilerParams(needs_layout_passes=False),
    )
    return f(coeff_t.reshape(-1), pack, cart_t.reshape(-1)).reshape(
        _NPARTS, 32, _N_NODES)


def _tc_stage2_body(acc_ref, scale_ref, mp_dis_ref, mp_cart_ref,
                    w1t_ref, b1_ref, w2t_ref, b2_ref, out_ref):
    a4 = acc_ref[...]
    acc = a4[0] + a4[1] + a4[2] + a4[3]
    sd = scale_ref[0:1, :]
    sc = scale_ref[1:2, :]

    md = (acc[0:8, :] + mp_dis_ref[...]) * sd
    h = jnp.dot(w1t_ref[...], md,
                preferred_element_type=jnp.float32) + b1_ref[...]
    h = h * (1.0 / (1.0 + jnp.exp(-h)))
    radial = jnp.dot(w2t_ref[...], h,
                     preferred_element_type=jnp.float32) + b2_ref[...]

    mc = (acc[8:32, :] + mp_cart_ref[...]) * sc
    x = mc[0:8, :]
    y = mc[8:16, :]
    z = mc[16:24, :]
    r2 = x * x + y * y + z * z
    s4 = x * y
    s5 = y * z
    s6 = 3.0 * z * z - r2
    s7 = x * z
    s8 = x * x - y * y
    ang2 = s4 * s4 + s5 * s5 + s6 * s6 + s7 * s7 + s8 * s8

    out_ref[0:8, :] = radial
    out_ref[8:16, :] = radial * r2
    out_ref[16:24, :] = radial * ang2


def _tc_stage2(acc, scale, mp_dis_t, mp_cart_f, w1t, b1c, w2t, b2c):
    return pl.pallas_call(
        _tc_stage2_body,
        out_shape=jax.ShapeDtypeStruct((24, _N_NODES), jnp.float32),
    )(acc, scale, mp_dis_t, mp_cart_f, w1t, b1c, w2t, b2c)


@jax.jit
def kernel(cart, cut_distances, iter_coeff, index_center, index_neigh,
           MP_dis, MP_cart, W1, b1, W2, b2):
    n = iter_coeff.shape[0]
    e = cart.shape[0]
    g = e // _CHUNK
    coeff_t = iter_coeff.T
    pidx = jnp.bitwise_or(index_neigh,
                          jnp.left_shift(index_center, 16))
    cutb = lax.bitcast_convert_type(cut_distances, jnp.int32)
    pack = jnp.stack([pidx.reshape(g, _CHUNK),
                      cutb.reshape(g, _CHUNK)], axis=1).reshape(-1)

    acc = _sc_scatter(coeff_t, pack, cart.T)

    scale = jnp.stack([iter_coeff[:, _NWAVE], iter_coeff[:, -1]], axis=0)
    mp_dis_t = MP_dis.T
    mp_cart_f = MP_cart.reshape(24, n)
    dens = _tc_stage2(acc, scale, mp_dis_t, mp_cart_f,
                      W1.T, b1[:, None], W2.T, b2[:, None])
    return dens.reshape(_MAX_L + 1, _NWAVE, n).transpose(2, 0, 1)
